# baseline (speedup 1.0000x reference)
import functools

import jax
import jax.numpy as jnp
from jax import lax
from jax.experimental import pallas as pl
from jax.experimental.pallas import tpu as pltpu
from jax.experimental.pallas import tpu_sc as plsc

N_GRAPHS = 16
GRID = 8
D = 128
N_NODES = 100000

NW = 32
CHUNK = 128
N_CHUNKS = 25
PER_W = CHUNK * N_CHUNKS
N_PAD = NW * PER_W
N_ALLOC = N_PAD + CHUNK
N_FULL = N_NODES // CHUNK
TAIL = N_NODES - N_FULL * CHUNK
N_ZPAD = N_ALLOC // CHUNK - N_FULL - 1
NBINS = N_GRAPHS * GRID * GRID
WBINS = 2 * GRID * GRID
ROWS_PER_TILE = NBINS // 16

_mesh = plsc.VectorSubcoreMesh(core_axis_name="c", subcore_axis_name="s")


@functools.partial(
    pl.kernel,
    out_type=jax.ShapeDtypeStruct((2, NBINS, D), jnp.float32),
    mesh=_mesh,
    compiler_params=pltpu.CompilerParams(needs_layout_passes=False),
    scratch_types=[
        pltpu.VMEM((CHUNK, D), jnp.float32),
        pltpu.VMEM((CHUNK, D), jnp.float32),
        pltpu.VMEM((CHUNK, 4), jnp.float32),
        pltpu.VMEM((CHUNK, 4), jnp.float32),
        pltpu.VMEM((CHUNK,), jnp.int32),
        pltpu.VMEM((CHUNK,), jnp.int32),
        pltpu.VMEM((CHUNK,), jnp.float32),
        pltpu.VMEM((WBINS, D), jnp.float32),
        pltpu.VMEM((ROWS_PER_TILE, D), jnp.float32),
        pltpu.VMEM_SHARED((NBINS, D), jnp.float32),
        pltpu.SemaphoreType.DMA,
        pltpu.SemaphoreType.DMA,
    ],
)
def _spp_scatter(feat, tailf, zpad, aux, out, feat_a, feat_b, aux_a, aux_b,
                 idx_a, idx_b, inv_c, lacc, bounce, acc, sem_fa, sem_fb):
    cid = lax.axis_index("c")
    sid = lax.axis_index("s")
    w = cid * 16 + sid
    base_w = w * PER_W

    iota16 = lax.broadcasted_iota(jnp.int32, (16,), 0)
    col0 = jnp.full((16,), 0, jnp.int32)
    col1 = jnp.full((16,), 1, jnp.int32)
    col2 = jnp.full((16,), 2, jnp.int32)
    col3 = jnp.full((16,), 3, jnp.int32)
    zero16 = jnp.zeros((16,), jnp.float32)

    def start_fetch(c, featb, auxb, sem):
        g = w * N_CHUNKS + c

        @pl.when(g < N_FULL)
        def _():
            pltpu.async_copy(feat.at[pl.ds(g * CHUNK, CHUNK)], featb, sem)

        @pl.when(g == N_FULL)
        def _():
            pltpu.async_copy(tailf.at[pl.ds(0, CHUNK)], featb, sem)

        @pl.when(g > N_FULL)
        def _():
            pltpu.async_copy(
                zpad.at[pl.ds((g - N_FULL - 1) * CHUNK, CHUNK)], featb, sem)

        pltpu.async_copy(aux.at[pl.ds(g * CHUNK, CHUNK)], auxb, sem)

    def wait_fetch(featb, auxb, sem):
        pltpu.make_async_copy(feat.at[pl.ds(0, CHUNK)], featb, sem).wait()
        pltpu.make_async_copy(aux.at[pl.ds(0, CHUNK)], auxb, sem).wait()

    def process(featb, auxb, idxb, wbase):
        def bin_body(i, carry):
            mn, mx = carry
            rows = iota16 + i * 16
            xv = plsc.load_gather(auxb, [rows, col0])
            yv = plsc.load_gather(auxb, [rows, col1])
            gv = plsc.load_gather(auxb, [rows, col3])
            cv = plsc.load_gather(auxb, [rows, col2])
            binv = (gv * float(GRID * GRID) + xv * float(GRID)
                    + yv).astype(jnp.int32)
            sl = pl.ds(i * 16, 16)
            idxb[sl] = binv
            inv_c[sl] = 1.0 / cv
            return jnp.minimum(mn, binv), jnp.maximum(mx, binv)

        big = jnp.full((16,), 1 << 30, jnp.int32)
        mn, mx = lax.fori_loop(0, CHUNK // 16, bin_body, (big, -big),
                               unroll=2)
        in_window = jnp.logical_and(jnp.min(mn) >= wbase,
                                    jnp.max(mx) < wbase + WBINS)
        wbase_v = jnp.full((16,), wbase, jnp.int32)

        @pl.when(in_window)
        def _():
            def node_local(j, _):
                jv = jnp.full((16,), j, jnp.int32)
                rowv = plsc.load_gather(idxb, [jv]) - wbase_v
                inv = plsc.load_gather(inv_c, [jv])
                for db in range(D // 16):
                    sl = pl.ds(db * 16, 16)
                    plsc.addupdate_scatter(
                        lacc, [rowv, iota16 + db * 16], featb[j, sl] * inv)
                return 0

            lax.fori_loop(0, CHUNK, node_local, 0, unroll=4)

        @pl.when(jnp.logical_not(in_window))
        def _():
            def node_body(j, _):
                inv = plsc.load_gather(inv_c, [jnp.full((16,), j, jnp.int32)])
                for db in range(D // 16):
                    sl = pl.ds(db * 16, 16)
                    featb[j, sl] = featb[j, sl] * inv
                return 0

            lax.fori_loop(0, CHUNK, node_body, 0, unroll=4)
            pltpu.sync_copy(featb, acc.at[idxb], add=True)

    start_fetch(0, feat_a, aux_a, sem_fa)

    def zero_row(r, _):
        for db in range(D // 16):
            bounce[r, pl.ds(db * 16, 16)] = zero16
        return 0

    lax.fori_loop(0, ROWS_PER_TILE, zero_row, 0)
    pltpu.sync_copy(bounce, acc.at[pl.ds(sid * ROWS_PER_TILE, ROWS_PER_TILE)])

    def zero_lacc(r, _):
        for db in range(D // 16):
            lacc[r, pl.ds(db * 16, 16)] = zero16
        return 0

    lax.fori_loop(0, WBINS, zero_lacc, 0)
    plsc.subcore_barrier()

    wait_fetch(feat_a, aux_a, sem_fa)
    g0 = plsc.load_gather(aux_a, [col0, col3])
    wbase = jnp.minimum(jnp.max(g0).astype(jnp.int32) * (GRID * GRID),
                        NBINS - WBINS)
    start_fetch(1, feat_b, aux_b, sem_fb)

    def pipe_body(k, _):
        ca = 2 * k
        process(feat_a, aux_a, idx_a, wbase)
        start_fetch(ca + 2, feat_a, aux_a, sem_fa)
        wait_fetch(feat_b, aux_b, sem_fb)
        process(feat_b, aux_b, idx_b, wbase)
        start_fetch(ca + 3, feat_b, aux_b, sem_fb)
        wait_fetch(feat_a, aux_a, sem_fa)
        return 0

    lax.fori_loop(0, (N_CHUNKS - 1) // 2, pipe_body, 0)

    process(feat_a, aux_a, idx_a, wbase)
    wait_fetch(feat_b, aux_b, sem_fb)

    def win_idx(i, _):
        idx_a[pl.ds(i * 16, 16)] = wbase + i * 16 + iota16
        return 0

    lax.fori_loop(0, WBINS // 16, win_idx, 0)
    pltpu.sync_copy(lacc, acc.at[idx_a], add=True)

    plsc.subcore_barrier()
    sl = pl.ds(sid * ROWS_PER_TILE, ROWS_PER_TILE)
    pltpu.sync_copy(acc.at[sl], bounce)
    pltpu.sync_copy(bounce, out.at[cid, sl])


def _add_partials_body(p_ref, o_ref):
    o_ref[...] = p_ref[0] + p_ref[1]


_add_partials = pl.pallas_call(
    _add_partials_body,
    out_shape=jax.ShapeDtypeStruct((NBINS, D), jnp.float32),
)


def kernel(features, xy, graph_ids):
    tailf = jnp.zeros((CHUNK, D), jnp.float32).at[:TAIL].set(
        features[N_FULL * CHUNK:])
    zpad = jnp.zeros((N_ZPAD * CHUNK, D), jnp.float32)
    pad = N_ALLOC - N_NODES
    aux = jnp.concatenate([
        xy[:, :2].astype(jnp.float32),
        xy[:, 2:3].astype(jnp.float32),
        graph_ids[:, None].astype(jnp.float32),
    ], axis=1)
    aux_pad = jnp.tile(
        jnp.array([[0.0, 0.0, 1.0, 0.0]], jnp.float32), (pad, 1))
    auxp = jnp.concatenate([aux, aux_pad], axis=0)
    partials = _spp_scatter(features, tailf, zpad, auxp)
    out = _add_partials(partials)
    return out.reshape(N_GRAPHS, GRID, GRID, D)


# speedup vs baseline: 1.7289x; 1.7289x over previous
import functools

import jax
import jax.numpy as jnp
from jax import lax
from jax.experimental import pallas as pl
from jax.experimental.pallas import tpu as pltpu
from jax.experimental.pallas import tpu_sc as plsc

N_GRAPHS = 16
GRID = 8
D = 128
N_NODES = 100000

NW = 32
CHUNK = 128
N_CHUNKS = 25
PER_W = CHUNK * N_CHUNKS
N_PAD = NW * PER_W
N_ALLOC = N_PAD + 2 * CHUNK
N_FULL = N_NODES // CHUNK
TAIL = N_NODES - N_FULL * CHUNK
N_ZPAD = N_ALLOC // CHUNK - N_FULL - 1
NBINS = N_GRAPHS * GRID * GRID
ROWS_PER_TILE = NBINS // 16

_mesh = plsc.VectorSubcoreMesh(core_axis_name="c", subcore_axis_name="s")


@functools.partial(
    pl.kernel,
    out_type=jax.ShapeDtypeStruct((2, NBINS, D), jnp.float32),
    mesh=_mesh,
    compiler_params=pltpu.CompilerParams(needs_layout_passes=False),
    scratch_types=[
        pltpu.VMEM((CHUNK, D), jnp.float32),
        pltpu.VMEM((CHUNK, D), jnp.float32),
        pltpu.VMEM((CHUNK, D), jnp.float32),
        pltpu.VMEM((CHUNK, 4), jnp.float32),
        pltpu.VMEM((CHUNK, 4), jnp.float32),
        pltpu.VMEM((CHUNK, 4), jnp.float32),
        pltpu.VMEM((CHUNK,), jnp.int32),
        pltpu.VMEM((CHUNK,), jnp.int32),
        pltpu.VMEM((CHUNK,), jnp.int32),
        pltpu.VMEM((CHUNK,), jnp.float32),
        pltpu.VMEM((ROWS_PER_TILE, D), jnp.float32),
        pltpu.VMEM_SHARED((NBINS, D), jnp.float32),
        pltpu.SemaphoreType.DMA,
        pltpu.SemaphoreType.DMA,
        pltpu.SemaphoreType.DMA,
        pltpu.SemaphoreType.DMA,
        pltpu.SemaphoreType.DMA,
        pltpu.SemaphoreType.DMA,
    ],
)
def _spp_scatter(feat, tailf, zpad, aux, out, feat_a, feat_b, feat_c,
                 aux_a, aux_b, aux_c, idx_a, idx_b, idx_c, inv_c, bounce,
                 acc, sem_fa, sem_fb, sem_fc, sem_sa, sem_sb, sem_sc):
    cid = lax.axis_index("c")
    sid = lax.axis_index("s")
    w = cid * 16 + sid

    iota16 = lax.broadcasted_iota(jnp.int32, (16,), 0)
    col0 = jnp.full((16,), 0, jnp.int32)
    col1 = jnp.full((16,), 1, jnp.int32)
    col2 = jnp.full((16,), 2, jnp.int32)
    col3 = jnp.full((16,), 3, jnp.int32)
    zero16 = jnp.zeros((16,), jnp.float32)

    BA = (feat_a, aux_a, idx_a, sem_fa, sem_sa)
    BB = (feat_b, aux_b, idx_b, sem_fb, sem_sb)
    BC = (feat_c, aux_c, idx_c, sem_fc, sem_sc)

    def start_fetch(c, buf):
        featb, auxb, _, sem, _ = buf
        g = w * N_CHUNKS + c

        @pl.when(g < N_FULL)
        def _():
            pltpu.async_copy(feat.at[pl.ds(g * CHUNK, CHUNK)], featb, sem)

        @pl.when(g == N_FULL)
        def _():
            pltpu.async_copy(tailf.at[pl.ds(0, CHUNK)], featb, sem)

        @pl.when(g > N_FULL)
        def _():
            pltpu.async_copy(
                zpad.at[pl.ds((g - N_FULL - 1) * CHUNK, CHUNK)], featb, sem)

        pltpu.async_copy(aux.at[pl.ds(g * CHUNK, CHUNK)], auxb, sem)

    def wait_fetch(buf):
        featb, auxb, _, sem, _ = buf
        pltpu.make_async_copy(feat.at[pl.ds(0, CHUNK)], featb, sem).wait()
        pltpu.make_async_copy(aux.at[pl.ds(0, CHUNK)], auxb, sem).wait()

    def start_scatter(buf):
        featb, _, idxb, _, sem = buf
        pltpu.async_copy(featb, acc.at[idxb], sem, add=True)

    def wait_scatter(buf):
        featb, _, idxb, _, sem = buf
        pltpu.make_async_copy(featb, acc.at[idxb], sem).wait()

    def process(buf):
        featb, auxb, idxb, _, _ = buf

        def bin_body(i, _):
            rows = iota16 + i * 16
            xv = plsc.load_gather(auxb, [rows, col0])
            yv = plsc.load_gather(auxb, [rows, col1])
            gv = plsc.load_gather(auxb, [rows, col3])
            cv = plsc.load_gather(auxb, [rows, col2])
            binv = gv * float(GRID * GRID) + xv * float(GRID) + yv
            sl = pl.ds(i * 16, 16)
            idxb[sl] = binv.astype(jnp.int32)
            inv_c[sl] = 1.0 / cv
            return 0

        lax.fori_loop(0, CHUNK // 16, bin_body, 0, unroll=2)

        def node_body(j, _):
            inv = plsc.load_gather(inv_c, [jnp.full((16,), j, jnp.int32)])
            for db in range(D // 16):
                sl = pl.ds(db * 16, 16)
                featb[j, sl] = featb[j, sl] * inv
            return 0

        lax.fori_loop(0, CHUNK, node_body, 0, unroll=4)

    def phase(c, cur, prev):
        wait_fetch(cur)
        process(cur)
        wait_scatter(prev)
        start_fetch(c + 2, prev)
        start_scatter(cur)

    start_fetch(0, BA)
    start_fetch(1, BB)

    def zero_row(r, _):
        for db in range(D // 16):
            bounce[r, pl.ds(db * 16, 16)] = zero16
        return 0

    lax.fori_loop(0, ROWS_PER_TILE, zero_row, 0)
    pltpu.sync_copy(bounce, acc.at[pl.ds(sid * ROWS_PER_TILE, ROWS_PER_TILE)])
    plsc.subcore_barrier()

    wait_fetch(BA)
    process(BA)
    start_fetch(2, BC)
    start_scatter(BA)

    def pipe_body(k, _):
        cb = 3 * k
        phase(cb + 1, BB, BA)
        phase(cb + 2, BC, BB)
        phase(cb + 3, BA, BC)
        return 0

    lax.fori_loop(0, (N_CHUNKS - 1) // 3, pipe_body, 0)

    wait_scatter(BA)
    wait_fetch(BB)
    wait_fetch(BC)

    plsc.subcore_barrier()
    sl = pl.ds(sid * ROWS_PER_TILE, ROWS_PER_TILE)
    pltpu.sync_copy(acc.at[sl], bounce)
    pltpu.sync_copy(bounce, out.at[cid, sl])


def _add_partials_body(p_ref, o_ref):
    o_ref[...] = p_ref[0] + p_ref[1]


_add_partials = pl.pallas_call(
    _add_partials_body,
    out_shape=jax.ShapeDtypeStruct((NBINS, D), jnp.float32),
)


def kernel(features, xy, graph_ids):
    tailf = jnp.zeros((CHUNK, D), jnp.float32).at[:TAIL].set(
        features[N_FULL * CHUNK:])
    zpad = jnp.zeros((N_ZPAD * CHUNK, D), jnp.float32)
    pad = N_ALLOC - N_NODES
    aux = jnp.concatenate([
        xy[:, :2].astype(jnp.float32),
        xy[:, 2:3].astype(jnp.float32),
        graph_ids[:, None].astype(jnp.float32),
    ], axis=1)
    aux_pad = jnp.tile(
        jnp.array([[0.0, 0.0, 1.0, 0.0]], jnp.float32), (pad, 1))
    auxp = jnp.concatenate([aux, aux_pad], axis=0)
    partials = _spp_scatter(features, tailf, zpad, auxp)
    out = _add_partials(partials)
    return out.reshape(N_GRAPHS, GRID, GRID, D)


# speedup vs baseline: 1.7702x; 1.0239x over previous
import functools

import jax
import jax.numpy as jnp
from jax import lax
from jax.experimental import pallas as pl
from jax.experimental.pallas import tpu as pltpu
from jax.experimental.pallas import tpu_sc as plsc

N_GRAPHS = 16
GRID = 8
D = 128
N_NODES = 100000

NW = 32
CHUNK = 128
N_CHUNKS = 25
PER_W = CHUNK * N_CHUNKS
N_FULL = N_NODES // CHUNK
TAIL = N_NODES - N_FULL * CHUNK
NBINS = N_GRAPHS * GRID * GRID
ROWS_PER_TILE = NBINS // 16

_mesh = plsc.VectorSubcoreMesh(core_axis_name="c", subcore_axis_name="s")


@functools.partial(
    pl.kernel,
    out_type=jax.ShapeDtypeStruct((2, NBINS, D), jnp.float32),
    mesh=_mesh,
    compiler_params=pltpu.CompilerParams(needs_layout_passes=False),
    scratch_types=[
        pltpu.VMEM((CHUNK, D), jnp.float32),
        pltpu.VMEM((CHUNK, D), jnp.float32),
        pltpu.VMEM((CHUNK, D), jnp.float32),
        pltpu.VMEM((CHUNK, 3), jnp.int32),
        pltpu.VMEM((CHUNK, 3), jnp.int32),
        pltpu.VMEM((CHUNK, 3), jnp.int32),
        pltpu.VMEM((CHUNK,), jnp.int32),
        pltpu.VMEM((CHUNK,), jnp.int32),
        pltpu.VMEM((CHUNK,), jnp.int32),
        pltpu.VMEM((CHUNK,), jnp.int32),
        pltpu.VMEM((CHUNK,), jnp.int32),
        pltpu.VMEM((CHUNK,), jnp.int32),
        pltpu.VMEM((CHUNK,), jnp.float32),
        pltpu.VMEM((ROWS_PER_TILE, D), jnp.float32),
        pltpu.VMEM_SHARED((NBINS, D), jnp.float32),
        pltpu.SemaphoreType.DMA,
        pltpu.SemaphoreType.DMA,
        pltpu.SemaphoreType.DMA,
        pltpu.SemaphoreType.DMA,
        pltpu.SemaphoreType.DMA,
        pltpu.SemaphoreType.DMA,
    ],
)
def _spp_scatter(feat, xy, gid, tail_feat, tail_xy, tail_gid, out,
                 feat_a, feat_b, feat_c, xy_a, xy_b, xy_c, gid_a, gid_b,
                 gid_c, idx_a, idx_b, idx_c, inv_c, bounce, acc,
                 sem_fa, sem_fb, sem_fc, sem_sa, sem_sb, sem_sc):
    cid = lax.axis_index("c")
    sid = lax.axis_index("s")
    w = cid * 16 + sid

    iota16 = lax.broadcasted_iota(jnp.int32, (16,), 0)
    col0 = jnp.full((16,), 0, jnp.int32)
    col1 = jnp.full((16,), 1, jnp.int32)
    col2 = jnp.full((16,), 2, jnp.int32)
    zero16 = jnp.zeros((16,), jnp.float32)

    BA = (feat_a, xy_a, gid_a, idx_a, sem_fa, sem_sa)
    BB = (feat_b, xy_b, gid_b, idx_b, sem_fb, sem_sb)
    BC = (feat_c, xy_c, gid_c, idx_c, sem_fc, sem_sc)

    def start_fetch(c, buf):
        featb, xyb, gidb, _, sem, _ = buf
        g = w * N_CHUNKS + c

        @pl.when(g < N_FULL)
        def _():
            base = g * CHUNK
            pltpu.async_copy(feat.at[pl.ds(base, CHUNK)], featb, sem)
            pltpu.async_copy(xy.at[pl.ds(base, CHUNK)], xyb, sem)
            pltpu.async_copy(gid.at[pl.ds(base, CHUNK)], gidb, sem)

        @pl.when(g >= N_FULL)
        def _():
            off = jnp.where(g == N_FULL, 0, CHUNK)
            pltpu.async_copy(tail_feat.at[pl.ds(off, CHUNK)], featb, sem)
            pltpu.async_copy(tail_xy.at[pl.ds(off, CHUNK)], xyb, sem)
            pltpu.async_copy(tail_gid.at[pl.ds(off, CHUNK)], gidb, sem)

    def wait_fetch(buf):
        featb, xyb, gidb, _, sem, _ = buf
        pltpu.make_async_copy(feat.at[pl.ds(0, CHUNK)], featb, sem).wait()
        pltpu.make_async_copy(xy.at[pl.ds(0, CHUNK)], xyb, sem).wait()
        pltpu.make_async_copy(gid.at[pl.ds(0, CHUNK)], gidb, sem).wait()

    def start_scatter(buf):
        featb, _, _, idxb, _, sem = buf
        pltpu.async_copy(featb, acc.at[idxb], sem, add=True)

    def wait_scatter(buf):
        featb, _, _, idxb, _, sem = buf
        pltpu.make_async_copy(featb, acc.at[idxb], sem).wait()

    def process(buf):
        featb, xyb, gidb, idxb, _, _ = buf

        def bin_body(i, _):
            rows = iota16 + i * 16
            xv = plsc.load_gather(xyb, [rows, col0])
            yv = plsc.load_gather(xyb, [rows, col1])
            cv = plsc.load_gather(xyb, [rows, col2])
            sl = pl.ds(i * 16, 16)
            idxb[sl] = gidb[sl] * (GRID * GRID) + xv * GRID + yv
            inv_c[sl] = 1.0 / cv.astype(jnp.float32)
            return 0

        lax.fori_loop(0, CHUNK // 16, bin_body, 0, unroll=2)

        def node_body(j, _):
            inv = plsc.load_gather(inv_c, [jnp.full((16,), j, jnp.int32)])
            for db in range(D // 16):
                sl = pl.ds(db * 16, 16)
                featb[j, sl] = featb[j, sl] * inv
            return 0

        lax.fori_loop(0, CHUNK, node_body, 0, unroll=8)

    def phase(c, cur, prev):
        wait_fetch(cur)
        process(cur)
        wait_scatter(prev)
        start_fetch(c + 2, prev)
        start_scatter(cur)

    start_fetch(0, BA)
    start_fetch(1, BB)

    def zero_row(r, _):
        for db in range(D // 16):
            bounce[r, pl.ds(db * 16, 16)] = zero16
        return 0

    lax.fori_loop(0, ROWS_PER_TILE, zero_row, 0)
    pltpu.sync_copy(bounce, acc.at[pl.ds(sid * ROWS_PER_TILE, ROWS_PER_TILE)])
    plsc.subcore_barrier()

    wait_fetch(BA)
    process(BA)
    start_fetch(2, BC)
    start_scatter(BA)

    def pipe_body(k, _):
        cb = 3 * k
        phase(cb + 1, BB, BA)
        phase(cb + 2, BC, BB)
        phase(cb + 3, BA, BC)
        return 0

    lax.fori_loop(0, (N_CHUNKS - 1) // 3, pipe_body, 0)

    wait_scatter(BA)
    wait_fetch(BB)
    wait_fetch(BC)

    plsc.subcore_barrier()
    sl = pl.ds(sid * ROWS_PER_TILE, ROWS_PER_TILE)
    pltpu.sync_copy(acc.at[sl], bounce)
    pltpu.sync_copy(bounce, out.at[cid, sl])


def _add_partials_body(p_ref, o_ref):
    o_ref[...] = p_ref[0] + p_ref[1]


_add_partials = pl.pallas_call(
    _add_partials_body,
    out_shape=jax.ShapeDtypeStruct((NBINS, D), jnp.float32),
)


def kernel(features, xy, graph_ids):
    cut = N_FULL * CHUNK
    tail_feat = jnp.zeros((2 * CHUNK, D), jnp.float32).at[:TAIL].set(
        features[cut:])
    xy32 = xy.astype(jnp.int32)
    gid32 = graph_ids.astype(jnp.int32)
    tail_xy = jnp.tile(
        jnp.array([[0, 0, 1]], jnp.int32), (2 * CHUNK, 1)).at[:TAIL].set(
        xy32[cut:])
    tail_gid = jnp.zeros((2 * CHUNK,), jnp.int32).at[:TAIL].set(gid32[cut:])
    partials = _spp_scatter(features, xy32, gid32, tail_feat, tail_xy,
                            tail_gid)
    out = _add_partials(partials)
    return out.reshape(N_GRAPHS, GRID, GRID, D)
